# speedup vs baseline: 1.0003x; 1.0003x over previous
import functools

import jax
import jax.numpy as jnp
from jax.experimental import pallas as pl

NUM_USERS = 50000
NUM_ITEMS = 50000
N = NUM_USERS + NUM_ITEMS
EMB = 32
E = 1600000

_BLK = 8000


def _combine_body(side_ref, ego_ref, wg_ref, bg_ref, wb_ref, bb_ref, out_ref):
    side = side_ref[...]
    ego = ego_ref[...]
    s = jnp.dot(side, wg_ref[...], preferred_element_type=jnp.float32) + bg_ref[...]
    s = jnp.where(s >= 0, s, 0.01 * s)
    b = jnp.dot(ego * side, wb_ref[...], preferred_element_type=jnp.float32) + bb_ref[...]
    b = jnp.where(b >= 0, b, 0.01 * b)
    out_ref[...] = s + b


def _combine(side, ego, Wg, bg, Wb, bb):
    n = side.shape[0]
    grid = (n + _BLK - 1) // _BLK
    return pl.pallas_call(
        _combine_body,
        grid=(grid,),
        in_specs=[
            pl.BlockSpec((_BLK, EMB), lambda i: (i, 0)),
            pl.BlockSpec((_BLK, EMB), lambda i: (i, 0)),
            pl.BlockSpec((EMB, EMB), lambda i: (0, 0)),
            pl.BlockSpec((1, EMB), lambda i: (0, 0)),
            pl.BlockSpec((EMB, EMB), lambda i: (0, 0)),
            pl.BlockSpec((1, EMB), lambda i: (0, 0)),
        ],
        out_specs=pl.BlockSpec((_BLK, EMB), lambda i: (i, 0)),
        out_shape=jax.ShapeDtypeStruct((n, EMB), jnp.float32),
    )(side, ego, Wg.T, bg.reshape(1, EMB), Wb.T, bb.reshape(1, EMB))


def kernel(user_indices, item_indices, adj_indices, adj_values, user_emb,
           item_emb, W_gc0, b_gc0, W_bi0, b_bi0, W_gc1, b_gc1, W_bi1, b_bi1):
    u_emb = jnp.take(user_emb, user_indices, axis=0)
    i_emb = jnp.take(item_emb, item_indices, axis=0)
    ego = jnp.concatenate([u_emb, i_emb], axis=0)
    src = adj_indices[0]
    dst = adj_indices[1]
    embs = [ego]
    for (Wg, bg, Wb, bb) in ((W_gc0, b_gc0, W_bi0, b_bi0),
                             (W_gc1, b_gc1, W_bi1, b_bi1)):
        gathered = jnp.take(ego, src, axis=0) * adj_values[:, None]
        side = jax.ops.segment_sum(gathered, dst, num_segments=N)
        ego = _combine(side, ego, Wg, bg, Wb, bb)
        embs.append(ego)
    all_c = jnp.concatenate(embs, axis=1)
    return (all_c[:NUM_USERS], all_c[NUM_USERS:])


# speedup vs baseline: 3.6248x; 3.6239x over previous
import functools

import jax
import jax.numpy as jnp
from jax import lax
from jax.experimental import pallas as pl
from jax.experimental.pallas import tpu as pltpu
from jax.experimental.pallas import tpu_sc as plsc

NUM_USERS = 50000
NUM_ITEMS = 50000
N = NUM_USERS + NUM_ITEMS
EMB = 32
HALF = 16
E = 1600000

NS = 16
EP = 1638400
EPT = EP // NS
C = 128
NCHUNK = EPT // C
NP = 100096
RPT = NP // NS
ZR = RPT // 17

_BLK = 1024


def _sc_propagate_body(ego_ref, src_ref, dst_ref, val_ref, side_ref,
                       idx_s, idx_d, vals_v, rows, zbuf, acc, sem):
    c = lax.axis_index("c")
    s = lax.axis_index("s")

    zero16 = jnp.zeros((HALF,), jnp.float32)

    def zrow(i, carry):
        zbuf[i, :] = zero16
        return carry
    lax.fori_loop(0, ZR, zrow, 0)

    row0 = s * RPT

    def zcp(k, carry):
        r0 = pl.multiple_of(row0 + k * ZR, 8)
        pltpu.sync_copy(zbuf, acc.at[pl.ds(r0, ZR)])
        return carry
    lax.fori_loop(0, RPT // ZR, zcp, 0)
    plsc.subcore_barrier()

    ebase = s * EPT
    coff = c * NP

    def chunk(g, carry):
        base = pl.multiple_of(ebase + g * C, C)
        pltpu.sync_copy(src_ref.at[pl.ds(base, C)], idx_s)
        pltpu.sync_copy(dst_ref.at[pl.ds(base, C)], idx_d)
        pltpu.sync_copy(val_ref.at[pl.ds(base, C)], vals_v)

        def adj(h, cr):
            o = pl.multiple_of(h * HALF, HALF)
            v = idx_s[pl.ds(o, HALF)]
            idx_s[pl.ds(o, HALF)] = v + coff
            return cr
        lax.fori_loop(0, C // HALF, adj, 0, unroll=True)

        pltpu.async_copy(ego_ref.at[idx_s], rows, sem).wait()

        def scale(h, cr):
            o = pl.multiple_of(h * HALF, HALF)
            vv = vals_v[pl.ds(o, HALF)]
            for j in range(HALF):
                e = o + j
                rows[e, :] = rows[e, :] * vv[j]
            return cr
        lax.fori_loop(0, C // HALF, scale, 0)

        pltpu.sync_copy(rows, acc.at[idx_d], add=True)
        return carry
    lax.fori_loop(0, NCHUNK, chunk, 0)
    plsc.subcore_barrier()

    def wb(k, carry):
        r0 = pl.multiple_of(row0 + k * ZR, 8)
        pltpu.sync_copy(acc.at[pl.ds(r0, ZR)], zbuf)
        pltpu.sync_copy(zbuf, side_ref.at[pl.ds(coff + r0, ZR)])
        return carry
    lax.fori_loop(0, RPT // ZR, wb, 0)


_sc_propagate = pl.kernel(
    _sc_propagate_body,
    out_type=jax.ShapeDtypeStruct((2 * NP, HALF), jnp.float32),
    mesh=plsc.VectorSubcoreMesh(core_axis_name="c", subcore_axis_name="s"),
    compiler_params=pltpu.CompilerParams(use_tc_tiling_on_sc=False),
    scratch_types=[
        pltpu.VMEM((C,), jnp.int32),
        pltpu.VMEM((C,), jnp.int32),
        pltpu.VMEM((C,), jnp.float32),
        pltpu.VMEM((C, HALF), jnp.float32),
        pltpu.VMEM((ZR, HALF), jnp.float32),
        pltpu.VMEM_SHARED((NP, HALF), jnp.float32),
        pltpu.SemaphoreType.DMA,
    ],
)


def _combine_body(side2_ref, ego2_ref, wg_ref, bg_ref, wb_ref, bb_ref,
                  out2_ref, outf_ref):
    side = jnp.concatenate([side2_ref[0], side2_ref[1]], axis=1)
    ego = jnp.concatenate([ego2_ref[0], ego2_ref[1]], axis=1)
    s = jnp.dot(side, wg_ref[...], preferred_element_type=jnp.float32) + bg_ref[...]
    s = jnp.where(s >= 0, s, 0.01 * s)
    b = jnp.dot(ego * side, wb_ref[...], preferred_element_type=jnp.float32) + bb_ref[...]
    b = jnp.where(b >= 0, b, 0.01 * b)
    res = s + b
    outf_ref[...] = res
    out2_ref[0] = res[:, :HALF]
    out2_ref[1] = res[:, HALF:]


def _combine(side2, ego2, Wg, bg, Wb, bb):
    grid = (NP + _BLK - 1) // _BLK
    return pl.pallas_call(
        _combine_body,
        grid=(grid,),
        in_specs=[
            pl.BlockSpec((2, _BLK, HALF), lambda i: (0, i, 0)),
            pl.BlockSpec((2, _BLK, HALF), lambda i: (0, i, 0)),
            pl.BlockSpec((EMB, EMB), lambda i: (0, 0)),
            pl.BlockSpec((1, EMB), lambda i: (0, 0)),
            pl.BlockSpec((EMB, EMB), lambda i: (0, 0)),
            pl.BlockSpec((1, EMB), lambda i: (0, 0)),
        ],
        out_specs=[
            pl.BlockSpec((2, _BLK, HALF), lambda i: (0, i, 0)),
            pl.BlockSpec((_BLK, EMB), lambda i: (i, 0)),
        ],
        out_shape=[
            jax.ShapeDtypeStruct((2, NP, HALF), jnp.float32),
            jax.ShapeDtypeStruct((NP, EMB), jnp.float32),
        ],
    )(side2, ego2, Wg.T, bg.reshape(1, EMB), Wb.T, bb.reshape(1, EMB))


def kernel(user_indices, item_indices, adj_indices, adj_values, user_emb,
           item_emb, W_gc0, b_gc0, W_bi0, b_bi0, W_gc1, b_gc1, W_bi1, b_bi1):
    u_emb = jnp.take(user_emb, user_indices, axis=0)
    i_emb = jnp.take(item_emb, item_indices, axis=0)
    ego0_flat = jnp.concatenate([u_emb, i_emb], axis=0)
    zpad = jnp.zeros((NP - N, HALF), jnp.float32)
    ego_cat = jnp.concatenate(
        [u_emb[:, :HALF], i_emb[:, :HALF], zpad,
         u_emb[:, HALF:], i_emb[:, HALF:], zpad],
        axis=0)

    pad = EP - E
    src = jnp.pad(adj_indices[0], (0, pad))
    dst = jnp.pad(adj_indices[1], (0, pad))
    vals = jnp.pad(adj_values, (0, pad))

    ego2 = ego_cat.reshape(2, NP, HALF)
    flats = [ego0_flat]
    for (Wg, bg, Wb, bb) in ((W_gc0, b_gc0, W_bi0, b_bi0),
                             (W_gc1, b_gc1, W_bi1, b_bi1)):
        side_cat = _sc_propagate(ego2.reshape(2 * NP, HALF), src, dst, vals)
        side2 = side_cat.reshape(2, NP, HALF)
        ego2, ego_flat = _combine(side2, ego2, Wg, bg, Wb, bb)
        flats.append(ego_flat[:N])
    all_c = jnp.concatenate(flats, axis=1)
    return (all_c[:NUM_USERS], all_c[NUM_USERS:])


# speedup vs baseline: 7.4592x; 2.0578x over previous
import functools

import jax
import jax.numpy as jnp
from jax import lax
from jax.experimental import pallas as pl
from jax.experimental.pallas import tpu as pltpu
from jax.experimental.pallas import tpu_sc as plsc

NUM_USERS = 50000
NUM_ITEMS = 50000
N = NUM_USERS + NUM_ITEMS
EMB = 32
HALF = 16
E = 1600000

NS = 16
EP = 1638400
EPT = EP // NS
C = 128
NCHUNK = EPT // C
NP = 100096
RPT = NP // NS
ZR = RPT // 17

_BLK = 1024


def _sc_propagate_body(ego_ref, pk_ref, val_ref, side_ref,
                       i0, i1, d0, d1, v0, v1, r0b, r1b, zbuf, acc,
                       si0, si1, sv0, sv1, sg0, sg1, ss0, ss1):
    c = lax.axis_index("c")
    s = lax.axis_index("s")

    zero16 = jnp.zeros((HALF,), jnp.float32)

    def zrow(i, carry):
        zbuf[i, :] = zero16
        return carry
    lax.fori_loop(0, ZR, zrow, 0)

    row0 = s * RPT

    def zcp(k, carry):
        rr = pl.multiple_of(row0 + k * ZR, 8)
        pltpu.sync_copy(zbuf, acc.at[pl.ds(rr, ZR)])
        return carry
    lax.fori_loop(0, RPT // ZR, zcp, 0)
    plsc.subcore_barrier()

    coff = c * NP
    cbase = s * NCHUNK

    def idx_start(ci, ib, vb, sem, vsem):
        pltpu.async_copy(pk_ref.at[ci], ib, sem)
        pltpu.async_copy(val_ref.at[ci], vb, vsem)

    def idx_wait(ib, vb, sem, vsem):
        pltpu.make_async_copy(pk_ref.at[0], ib, sem).wait()
        pltpu.make_async_copy(val_ref.at[0], vb, vsem).wait()

    def adjust(ib):
        for h in range(C // HALF):
            o = h * HALF
            ib[0, pl.ds(o, HALF)] = ib[0, pl.ds(o, HALF)] + coff

    def gather_start(ib, rb, sem):
        pltpu.async_copy(ego_ref.at[ib.at[0]], rb, sem)

    def gather_wait(ib, rb, sem):
        pltpu.make_async_copy(ego_ref.at[ib.at[0]], rb, sem).wait()

    def dcopy_scale(ib, db, vb, rb):
        for h in range(C // HALF):
            o = h * HALF
            db[pl.ds(o, HALF)] = ib[1, pl.ds(o, HALF)]
            vv = vb[pl.ds(o, HALF)]
            for j in range(HALF):
                rb[o + j, :] = rb[o + j, :] * vv[j]

    def scatter_start(db, rb, sem):
        pltpu.async_copy(rb, acc.at[db], sem, add=True)

    def scatter_wait(db, rb, sem):
        pltpu.make_async_copy(rb, acc.at[db], sem).wait()

    idx_start(cbase + 0, i0, v0, si0, sv0)
    idx_start(cbase + 1, i1, v1, si1, sv1)
    idx_wait(i0, v0, si0, sv0)
    adjust(i0)
    gather_start(i0, r0b, sg0)
    idx_wait(i1, v1, si1, sv1)
    adjust(i1)
    gather_start(i1, r1b, sg1)
    gather_wait(i0, r0b, sg0)
    dcopy_scale(i0, d0, v0, r0b)
    scatter_start(d0, r0b, ss0)
    idx_start(cbase + 2, i0, v0, si0, sv0)
    gather_wait(i1, r1b, sg1)
    dcopy_scale(i1, d1, v1, r1b)
    scatter_start(d1, r1b, ss1)
    idx_start(cbase + 3, i1, v1, si1, sv1)

    def pair(g2, carry):
        e = cbase + 2 * g2
        idx_wait(i0, v0, si0, sv0)
        adjust(i0)
        scatter_wait(d0, r0b, ss0)
        gather_start(i0, r0b, sg0)
        idx_wait(i1, v1, si1, sv1)
        adjust(i1)
        scatter_wait(d1, r1b, ss1)
        gather_start(i1, r1b, sg1)
        gather_wait(i0, r0b, sg0)
        dcopy_scale(i0, d0, v0, r0b)
        scatter_start(d0, r0b, ss0)
        idx_start(e + 2, i0, v0, si0, sv0)
        gather_wait(i1, r1b, sg1)
        dcopy_scale(i1, d1, v1, r1b)
        scatter_start(d1, r1b, ss1)
        idx_start(e + 3, i1, v1, si1, sv1)
        return carry
    lax.fori_loop(1, NCHUNK // 2 - 1, pair, 0)

    idx_wait(i0, v0, si0, sv0)
    adjust(i0)
    scatter_wait(d0, r0b, ss0)
    gather_start(i0, r0b, sg0)
    idx_wait(i1, v1, si1, sv1)
    adjust(i1)
    scatter_wait(d1, r1b, ss1)
    gather_start(i1, r1b, sg1)
    gather_wait(i0, r0b, sg0)
    dcopy_scale(i0, d0, v0, r0b)
    scatter_start(d0, r0b, ss0)
    gather_wait(i1, r1b, sg1)
    dcopy_scale(i1, d1, v1, r1b)
    scatter_start(d1, r1b, ss1)
    scatter_wait(d0, r0b, ss0)
    scatter_wait(d1, r1b, ss1)
    plsc.subcore_barrier()

    def wb(k, carry):
        rr = pl.multiple_of(row0 + k * ZR, 8)
        pltpu.sync_copy(acc.at[pl.ds(rr, ZR)], zbuf)
        pltpu.sync_copy(zbuf, side_ref.at[pl.ds(coff + rr, ZR)])
        return carry
    lax.fori_loop(0, RPT // ZR, wb, 0)


_sc_propagate = pl.kernel(
    _sc_propagate_body,
    out_type=jax.ShapeDtypeStruct((2 * NP, HALF), jnp.float32),
    mesh=plsc.VectorSubcoreMesh(core_axis_name="c", subcore_axis_name="s"),
    compiler_params=pltpu.CompilerParams(use_tc_tiling_on_sc=False),
    scratch_types=[
        pltpu.VMEM((2, C), jnp.int32),
        pltpu.VMEM((2, C), jnp.int32),
        pltpu.VMEM((C,), jnp.int32),
        pltpu.VMEM((C,), jnp.int32),
        pltpu.VMEM((C,), jnp.float32),
        pltpu.VMEM((C,), jnp.float32),
        pltpu.VMEM((C, HALF), jnp.float32),
        pltpu.VMEM((C, HALF), jnp.float32),
        pltpu.VMEM((ZR, HALF), jnp.float32),
        pltpu.VMEM_SHARED((NP, HALF), jnp.float32),
        pltpu.SemaphoreType.DMA,
        pltpu.SemaphoreType.DMA,
        pltpu.SemaphoreType.DMA,
        pltpu.SemaphoreType.DMA,
        pltpu.SemaphoreType.DMA,
        pltpu.SemaphoreType.DMA,
        pltpu.SemaphoreType.DMA,
        pltpu.SemaphoreType.DMA,
    ],
)


def _combine_body(side2_ref, ego2_ref, wg_ref, bg_ref, wb_ref, bb_ref,
                  out2_ref, outf_ref):
    side = jnp.concatenate([side2_ref[0], side2_ref[1]], axis=1)
    ego = jnp.concatenate([ego2_ref[0], ego2_ref[1]], axis=1)
    s = jnp.dot(side, wg_ref[...], preferred_element_type=jnp.float32) + bg_ref[...]
    s = jnp.where(s >= 0, s, 0.01 * s)
    b = jnp.dot(ego * side, wb_ref[...], preferred_element_type=jnp.float32) + bb_ref[...]
    b = jnp.where(b >= 0, b, 0.01 * b)
    res = s + b
    outf_ref[...] = res
    out2_ref[0] = res[:, :HALF]
    out2_ref[1] = res[:, HALF:]


def _combine(side2, ego2, Wg, bg, Wb, bb):
    grid = (NP + _BLK - 1) // _BLK
    return pl.pallas_call(
        _combine_body,
        grid=(grid,),
        in_specs=[
            pl.BlockSpec((2, _BLK, HALF), lambda i: (0, i, 0)),
            pl.BlockSpec((2, _BLK, HALF), lambda i: (0, i, 0)),
            pl.BlockSpec((EMB, EMB), lambda i: (0, 0)),
            pl.BlockSpec((1, EMB), lambda i: (0, 0)),
            pl.BlockSpec((EMB, EMB), lambda i: (0, 0)),
            pl.BlockSpec((1, EMB), lambda i: (0, 0)),
        ],
        out_specs=[
            pl.BlockSpec((2, _BLK, HALF), lambda i: (0, i, 0)),
            pl.BlockSpec((_BLK, EMB), lambda i: (i, 0)),
        ],
        out_shape=[
            jax.ShapeDtypeStruct((2, NP, HALF), jnp.float32),
            jax.ShapeDtypeStruct((NP, EMB), jnp.float32),
        ],
    )(side2, ego2, Wg.T, bg.reshape(1, EMB), Wb.T, bb.reshape(1, EMB))


def kernel(user_indices, item_indices, adj_indices, adj_values, user_emb,
           item_emb, W_gc0, b_gc0, W_bi0, b_bi0, W_gc1, b_gc1, W_bi1, b_bi1):
    u_emb = jnp.take(user_emb, user_indices, axis=0)
    i_emb = jnp.take(item_emb, item_indices, axis=0)
    ego0_flat = jnp.concatenate([u_emb, i_emb], axis=0)
    zpad = jnp.zeros((NP - N, HALF), jnp.float32)
    ego_cat = jnp.concatenate(
        [u_emb[:, :HALF], i_emb[:, :HALF], zpad,
         u_emb[:, HALF:], i_emb[:, HALF:], zpad],
        axis=0)

    pad = EP - E
    src = jnp.pad(adj_indices[0], (0, pad))
    dst = jnp.pad(adj_indices[1], (0, pad))
    vals = jnp.pad(adj_values, (0, pad))
    packed = (jnp.stack([src, dst])
              .reshape(2, EP // C, C)
              .transpose(1, 0, 2))
    valsr = vals.reshape(EP // C, C)

    ego2 = ego_cat.reshape(2, NP, HALF)
    flats = [ego0_flat]
    for (Wg, bg, Wb, bb) in ((W_gc0, b_gc0, W_bi0, b_bi0),
                             (W_gc1, b_gc1, W_bi1, b_bi1)):
        side_cat = _sc_propagate(ego2.reshape(2 * NP, HALF), packed, valsr)
        side2 = side_cat.reshape(2, NP, HALF)
        ego2, ego_flat = _combine(side2, ego2, Wg, bg, Wb, bb)
        flats.append(ego_flat[:N])
    all_c = jnp.concatenate(flats, axis=1)
    return (all_c[:NUM_USERS], all_c[NUM_USERS:])


# speedup vs baseline: 9.5034x; 1.2740x over previous
import functools

import jax
import jax.numpy as jnp
from jax import lax
from jax.experimental import pallas as pl
from jax.experimental.pallas import tpu as pltpu
from jax.experimental.pallas import tpu_sc as plsc

NUM_USERS = 50000
NUM_ITEMS = 50000
N = NUM_USERS + NUM_ITEMS
EMB = 32
HALF = 16
E = 1600000

NS = 16
EP = 1638400
EPT = EP // NS
C = 128
NCHUNK = EPT // C
NQ = NCHUNK // 4
PQ = EP // (4 * C)
NP = 100096
RPT = NP // NS
ZR = RPT // 17

_BLK = 1024


def _sc_propagate_body(ego_ref, pk_ref, val_ref, side_ref,
                       q0, q1, vq0, vq1, r0, r1, r2, r3, zbuf, acc,
                       sqi0, sqi1, sqv0, sqv1,
                       sg0, sg1, sg2, sg3, ss0, ss1, ss2, ss3):
    c = lax.axis_index("c")
    s = lax.axis_index("s")
    Q, VQ = [q0, q1], [vq0, vq1]
    R = [r0, r1, r2, r3]
    SQI, SQV = [sqi0, sqi1], [sqv0, sqv1]
    SG, SS = [sg0, sg1, sg2, sg3], [ss0, ss1, ss2, ss3]

    zero16 = jnp.zeros((HALF,), jnp.float32)

    def zrow(i, carry):
        zbuf[i, :] = zero16
        return carry
    lax.fori_loop(0, ZR, zrow, 0)

    row0 = s * RPT

    def zcp(k, carry):
        rr = pl.multiple_of(row0 + k * ZR, 8)
        pltpu.sync_copy(zbuf, acc.at[pl.ds(rr, ZR)])
        return carry
    lax.fori_loop(0, RPT // ZR, zcp, 0)
    plsc.subcore_barrier()

    coff = c * NP
    qbase = s * NQ

    def qstart(qid, b):
        pltpu.async_copy(pk_ref.at[qid], Q[b], SQI[b])
        pltpu.async_copy(val_ref.at[qid], VQ[b], SQV[b])

    def qwait(b):
        pltpu.make_async_copy(pk_ref.at[0], Q[b], SQI[b]).wait()
        pltpu.make_async_copy(val_ref.at[0], VQ[b], SQV[b]).wait()

    def adjust(b, j):
        def adj(h, cr):
            o = pl.multiple_of(h * HALF, HALF)
            Q[b][j, 0, pl.ds(o, HALF)] = Q[b][j, 0, pl.ds(o, HALF)] + coff
            return cr
        lax.fori_loop(0, C // HALF, adj, 0)

    def gs(b, j, r):
        pltpu.async_copy(ego_ref.at[Q[b].at[j, 0]], R[r], SG[r])

    def gw(b, j, r):
        pltpu.make_async_copy(ego_ref.at[Q[b].at[j, 0]], R[r], SG[r]).wait()

    def scale(b, j, r):
        def sc16(h, cr):
            o = pl.multiple_of(h * HALF, HALF)
            vv = VQ[b][j, pl.ds(o, HALF)]
            for t in range(HALF):
                R[r][o + t, :] = R[r][o + t, :] * vv[t]
            return cr
        lax.fori_loop(0, C // HALF, sc16, 0)

    def st(b, j, r):
        pltpu.async_copy(R[r], acc.at[Q[b].at[j, 1]], SS[r], add=True)

    def sw(b, j, r):
        pltpu.make_async_copy(R[r], acc.at[Q[b].at[j, 1]], SS[r]).wait()

    qstart(qbase + 0, 0)
    qstart(qbase + 1, 1)
    qwait(0)
    adjust(0, 0)
    gs(0, 0, 0)
    adjust(0, 1)
    gs(0, 1, 1)

    adjust(0, 2)
    gs(0, 2, 2)
    adjust(0, 3)
    gs(0, 3, 3)
    gw(0, 0, 0); scale(0, 0, 0); st(0, 0, 0)
    gw(0, 1, 1); scale(0, 1, 1); st(0, 1, 1)
    qwait(1)
    sw(0, 0, 0); adjust(1, 0); gs(1, 0, 0)
    sw(0, 1, 1); adjust(1, 1); gs(1, 1, 1)
    gw(0, 2, 2); scale(0, 2, 2); st(0, 2, 2)
    gw(0, 3, 3); scale(0, 3, 3); st(0, 3, 3)
    qstart(qbase + 2, 0)

    def body(m_id, a, nb):
        sw(a, 2, 2); adjust(a, 2); gs(a, 2, 2)
        sw(a, 3, 3); adjust(a, 3); gs(a, 3, 3)
        gw(a, 0, 0); scale(a, 0, 0); st(a, 0, 0)
        gw(a, 1, 1); scale(a, 1, 1); st(a, 1, 1)
        qwait(nb)
        sw(a, 0, 0); adjust(nb, 0); gs(nb, 0, 0)
        sw(a, 1, 1); adjust(nb, 1); gs(nb, 1, 1)
        gw(a, 2, 2); scale(a, 2, 2); st(a, 2, 2)
        gw(a, 3, 3); scale(a, 3, 3); st(a, 3, 3)
        qstart(m_id + 2, a)

    def pair(p, carry):
        m1 = qbase + 2 * p + 1
        body(m1, 1, 0)
        body(m1 + 1, 0, 1)
        return carry
    lax.fori_loop(0, (NQ - 2) // 2, pair, 0)

    sw(1, 2, 2); adjust(1, 2); gs(1, 2, 2)
    sw(1, 3, 3); adjust(1, 3); gs(1, 3, 3)
    gw(1, 0, 0); scale(1, 0, 0); st(1, 0, 0)
    gw(1, 1, 1); scale(1, 1, 1); st(1, 1, 1)
    gw(1, 2, 2); scale(1, 2, 2); st(1, 2, 2)
    gw(1, 3, 3); scale(1, 3, 3); st(1, 3, 3)
    sw(1, 0, 0)
    sw(1, 1, 1)
    sw(1, 2, 2)
    sw(1, 3, 3)
    qwait(0)
    plsc.subcore_barrier()

    def wb(k, carry):
        rr = pl.multiple_of(row0 + k * ZR, 8)
        pltpu.sync_copy(acc.at[pl.ds(rr, ZR)], zbuf)
        pltpu.sync_copy(zbuf, side_ref.at[pl.ds(coff + rr, ZR)])
        return carry
    lax.fori_loop(0, RPT // ZR, wb, 0)


_sc_propagate = pl.kernel(
    _sc_propagate_body,
    out_type=jax.ShapeDtypeStruct((2 * NP, HALF), jnp.float32),
    mesh=plsc.VectorSubcoreMesh(core_axis_name="c", subcore_axis_name="s"),
    compiler_params=pltpu.CompilerParams(use_tc_tiling_on_sc=False),
    scratch_types=(
        [
            pltpu.VMEM((4, 2, C), jnp.int32),
            pltpu.VMEM((4, 2, C), jnp.int32),
            pltpu.VMEM((4, C), jnp.float32),
            pltpu.VMEM((4, C), jnp.float32),
        ]
        + [pltpu.VMEM((C, HALF), jnp.float32)] * 4
        + [
            pltpu.VMEM((ZR, HALF), jnp.float32),
            pltpu.VMEM_SHARED((NP, HALF), jnp.float32),
        ]
        + [pltpu.SemaphoreType.DMA] * 12
    ),
)


def _combine_body(side2_ref, ego2_ref, wg_ref, bg_ref, wb_ref, bb_ref,
                  out2_ref, outf_ref):
    side = jnp.concatenate([side2_ref[0], side2_ref[1]], axis=1)
    ego = jnp.concatenate([ego2_ref[0], ego2_ref[1]], axis=1)
    s = jnp.dot(side, wg_ref[...], preferred_element_type=jnp.float32) + bg_ref[...]
    s = jnp.where(s >= 0, s, 0.01 * s)
    b = jnp.dot(ego * side, wb_ref[...], preferred_element_type=jnp.float32) + bb_ref[...]
    b = jnp.where(b >= 0, b, 0.01 * b)
    res = s + b
    outf_ref[...] = res
    out2_ref[0] = res[:, :HALF]
    out2_ref[1] = res[:, HALF:]


def _combine(side2, ego2, Wg, bg, Wb, bb):
    grid = (NP + _BLK - 1) // _BLK
    return pl.pallas_call(
        _combine_body,
        grid=(grid,),
        in_specs=[
            pl.BlockSpec((2, _BLK, HALF), lambda i: (0, i, 0)),
            pl.BlockSpec((2, _BLK, HALF), lambda i: (0, i, 0)),
            pl.BlockSpec((EMB, EMB), lambda i: (0, 0)),
            pl.BlockSpec((1, EMB), lambda i: (0, 0)),
            pl.BlockSpec((EMB, EMB), lambda i: (0, 0)),
            pl.BlockSpec((1, EMB), lambda i: (0, 0)),
        ],
        out_specs=[
            pl.BlockSpec((2, _BLK, HALF), lambda i: (0, i, 0)),
            pl.BlockSpec((_BLK, EMB), lambda i: (i, 0)),
        ],
        out_shape=[
            jax.ShapeDtypeStruct((2, NP, HALF), jnp.float32),
            jax.ShapeDtypeStruct((NP, EMB), jnp.float32),
        ],
    )(side2, ego2, Wg.T, bg.reshape(1, EMB), Wb.T, bb.reshape(1, EMB))


def kernel(user_indices, item_indices, adj_indices, adj_values, user_emb,
           item_emb, W_gc0, b_gc0, W_bi0, b_bi0, W_gc1, b_gc1, W_bi1, b_bi1):
    u_emb = jnp.take(user_emb, user_indices, axis=0)
    i_emb = jnp.take(item_emb, item_indices, axis=0)
    ego0_flat = jnp.concatenate([u_emb, i_emb], axis=0)
    zpad = jnp.zeros((NP - N, HALF), jnp.float32)
    ego_cat = jnp.concatenate(
        [u_emb[:, :HALF], i_emb[:, :HALF], zpad,
         u_emb[:, HALF:], i_emb[:, HALF:], zpad],
        axis=0)

    pad = EP - E
    src = jnp.pad(adj_indices[0], (0, pad))
    dst = jnp.pad(adj_indices[1], (0, pad))
    vals = jnp.pad(adj_values, (0, pad))
    srcr = src.reshape(PQ, 4, C)
    dstr = dst.reshape(PQ, 4, C)
    packed = jnp.pad(jnp.stack([srcr, dstr], axis=2),
                     ((0, 1), (0, 0), (0, 0), (0, 0)))
    valsr = jnp.pad(vals.reshape(PQ, 4, C), ((0, 1), (0, 0), (0, 0)))

    ego2 = ego_cat.reshape(2, NP, HALF)
    flats = [ego0_flat]
    for (Wg, bg, Wb, bb) in ((W_gc0, b_gc0, W_bi0, b_bi0),
                             (W_gc1, b_gc1, W_bi1, b_bi1)):
        side_cat = _sc_propagate(ego2.reshape(2 * NP, HALF), packed, valsr)
        side2 = side_cat.reshape(2, NP, HALF)
        ego2, ego_flat = _combine(side2, ego2, Wg, bg, Wb, bb)
        flats.append(ego_flat[:N])
    all_c = jnp.concatenate(flats, axis=1)
    return (all_c[:NUM_USERS], all_c[NUM_USERS:])
